# Initial kernel scaffold; baseline (speedup 1.0000x reference)
import jax
import jax.numpy as jnp
from jax.experimental import pallas as pl


def kernel(embedding, adj, edge_vals, W_item0, W_item1, W_i1, W_i2, edge_index, channel):
    raise NotImplementedError("write your pallas kernel here")


# speedup vs baseline: 2.2711x; 2.2711x over previous
import functools

import jax
import jax.numpy as jnp
from jax import lax
from jax.experimental import pallas as pl
from jax.experimental.pallas import tpu as pltpu
from jax.experimental.pallas import tpu_sc as plsc

N = 50000
E = 800000
D = 100
K = 100
DP = 128
NP = 50176
CHUNK = 12544
RPT = CHUNK // 16
ZB = 112
NSC, NTEC = 2, 16
EPT = E // NTEC
BE = 2000
NB = EPT // BE
NV = BE // 16
MB = 2048

_f32 = jnp.float32
_i32 = jnp.int32


def _spmm_body(row_h, col_h, val_h, x_h, out_h,
               acc, rowv, colv, valv, mcol, mval, mlrow, rows_v, zb, gsem):
    c = lax.axis_index("c")
    s = lax.axis_index("s")

    def _zrow(i, _):
        for dblk in range(DP // 16):
            zb[i, pl.ds(dblk * 16, 16)] = jnp.zeros((16,), _f32)
        return 0
    lax.fori_loop(0, ZB, _zrow, 0)

    base = s * RPT
    for p in range(2):
        lo = (2 * c + p) * CHUNK

        for q in range(RPT // ZB):
            pltpu.sync_copy(zb, acc.at[pl.ds(base + q * ZB, ZB)])
        plsc.subcore_barrier()

        def _block(b, _):
            eb = s * EPT + b * BE
            pltpu.sync_copy(row_h.at[pl.ds(eb, BE)], rowv)
            pltpu.sync_copy(col_h.at[pl.ds(eb, BE)], colv)
            pltpu.sync_copy(val_h.at[pl.ds(eb, BE)], valv)

            def _scan(i, cnt):
                r = rowv[pl.ds(i * 16, 16)]
                ci = colv[pl.ds(i * 16, 16)]
                v = valv[pl.ds(i * 16, 16)]
                m = (r >= lo) & (r < lo + CHUNK)
                mi = m.astype(_i32)
                pos = cnt + plsc.cumsum(mi) - 1
                plsc.store_scatter(mcol, [pos], ci, mask=m)
                plsc.store_scatter(mval, [pos], v, mask=m)
                plsc.store_scatter(mlrow, [pos], r - lo, mask=m)
                return cnt + jnp.sum(mi)
            cnt = lax.fori_loop(0, NV, _scan, jnp.int32(0))

            cnt16 = ((cnt + 15) // 16) * 16
            lane = lax.iota(_i32, 16)
            padm = lane < (cnt16 - cnt)
            ppos = cnt + lane
            plsc.store_scatter(mcol, [ppos], jnp.zeros((16,), _i32), mask=padm)
            plsc.store_scatter(mval, [ppos], jnp.zeros((16,), _f32), mask=padm)
            plsc.store_scatter(mlrow, [ppos], jnp.zeros((16,), _i32), mask=padm)

            def _group(g, _):
                cv = mcol[pl.ds(g * 16, 16)]
                lv = mlrow[pl.ds(g * 16, 16)]
                vv = mval[pl.ds(g * 16, 16)]
                pltpu.async_copy(x_h.at[cv], rows_v, gsem).wait()
                for l in range(16):
                    vb = jnp.full((16,), vv[l], _f32)
                    for dblk in range(DP // 16):
                        sl = pl.ds(dblk * 16, 16)
                        rows_v[l, sl] = rows_v[l, sl] * vb
                pltpu.sync_copy(rows_v, acc.at[lv], add=True)
                return 0
            lax.fori_loop(0, cnt16 // 16, _group, 0)
            return 0
        lax.fori_loop(0, NB, _block, 0)
        plsc.subcore_barrier()

        for q in range(RPT // ZB):
            rs = base + q * ZB
            pltpu.sync_copy(acc.at[pl.ds(rs, ZB)], out_h.at[pl.ds(lo + rs, ZB)])
        plsc.subcore_barrier()


@functools.cache
def _spmm_call():
    return pl.kernel(
        _spmm_body,
        out_type=jax.ShapeDtypeStruct((NP, DP), _f32),
        mesh=plsc.VectorSubcoreMesh(core_axis_name="c", subcore_axis_name="s",
                                    num_cores=NSC, num_subcores=NTEC),
        compiler_params=pltpu.CompilerParams(needs_layout_passes=False),
        scratch_types=[
            pltpu.VMEM_SHARED((CHUNK, DP), _f32),
            pltpu.VMEM((BE,), _i32),
            pltpu.VMEM((BE,), _i32),
            pltpu.VMEM((BE,), _f32),
            pltpu.VMEM((MB,), _i32),
            pltpu.VMEM((MB,), _f32),
            pltpu.VMEM((MB,), _i32),
            pltpu.VMEM((16, DP), _f32),
            pltpu.VMEM((ZB, DP), _f32),
            pltpu.SemaphoreType.DMA,
        ],
    )


BN = 1024
GRID = NP // BN


def _mm_body(x_ref, w_ref, o_ref):
    o_ref[...] = jnp.dot(x_ref[...], w_ref[...], preferred_element_type=_f32)


def _matmul(x, w):
    return pl.pallas_call(
        _mm_body,
        grid=(GRID,),
        in_specs=[pl.BlockSpec((BN, DP), lambda i: (i, 0)),
                  pl.BlockSpec((DP, DP), lambda i: (0, 0))],
        out_specs=pl.BlockSpec((BN, DP), lambda i: (i, 0)),
        out_shape=jax.ShapeDtypeStruct((NP, DP), _f32),
    )(x, w)


def _cluster_body(a_ref, adjt_ref, wi1_ref, wi2_ref, h1_ref, h2_ref):
    i = pl.program_id(0)
    a = a_ref[...]
    t = jnp.maximum(jnp.dot(a, wi1_ref[...], preferred_element_type=_f32), 0.0)
    logits = jnp.dot(t, wi2_ref[...], preferred_element_type=_f32)
    lane = lax.broadcasted_iota(_i32, (BN, DP), 1)
    logits = jnp.where(lane < K, logits, -1e30)
    mx = jnp.max(logits, axis=1, keepdims=True)
    ex = jnp.exp(logits - mx)
    h1 = ex / jnp.sum(ex, axis=1, keepdims=True)
    h1_ref[...] = h1
    cc = h1 * adjt_ref[...]
    bb = cc / (jnp.sum(cc, axis=1, keepdims=True) + 1e-8)
    contrib = lax.dot_general(bb, a, (((0,), (0,)), ((), ())),
                              preferred_element_type=_f32)

    @pl.when(i == 0)
    def _():
        h2_ref[...] = jnp.zeros((DP, DP), _f32)

    h2_ref[...] += contrib


def _cluster(a, adjt, wi1, wi2):
    return pl.pallas_call(
        _cluster_body,
        grid=(GRID,),
        in_specs=[pl.BlockSpec((BN, DP), lambda i: (i, 0)),
                  pl.BlockSpec((BN, DP), lambda i: (i, 0)),
                  pl.BlockSpec((DP, DP), lambda i: (0, 0)),
                  pl.BlockSpec((DP, DP), lambda i: (0, 0))],
        out_specs=[pl.BlockSpec((BN, DP), lambda i: (i, 0)),
                   pl.BlockSpec((DP, DP), lambda i: (0, 0))],
        out_shape=[jax.ShapeDtypeStruct((NP, DP), _f32),
                   jax.ShapeDtypeStruct((DP, DP), _f32)],
    )(a, adjt, wi1, wi2)


def _l2(v):
    return v / jnp.maximum(jnp.sqrt(jnp.sum(v * v, axis=1, keepdims=True)), 1e-12)


def _post_body(h1_ref, h2_ref, a_ref, xn_ref, l2x_ref, l2h_ref):
    h3 = jnp.dot(h1_ref[...], h2_ref[...], preferred_element_type=_f32)
    xn = h3 + a_ref[...]
    xn_ref[...] = xn
    l2x_ref[...] = _l2(xn)
    l2h_ref[...] = _l2(h3)


def _post(h1, h2, a):
    return pl.pallas_call(
        _post_body,
        grid=(GRID,),
        in_specs=[pl.BlockSpec((BN, DP), lambda i: (i, 0)),
                  pl.BlockSpec((DP, DP), lambda i: (0, 0)),
                  pl.BlockSpec((BN, DP), lambda i: (i, 0))],
        out_specs=[pl.BlockSpec((BN, DP), lambda i: (i, 0))] * 3,
        out_shape=[jax.ShapeDtypeStruct((NP, DP), _f32)] * 3,
    )(h1, h2, a)


def _final_body(h1_ref, h2_ref, a_ref, x0_ref, l2x1_ref, l2h1_ref,
                item_ref, hs_ref):
    h3 = jnp.dot(h1_ref[...], h2_ref[...], preferred_element_type=_f32)
    xn = h3 + a_ref[...]
    item_ref[...] = (x0_ref[...] + l2x1_ref[...] + _l2(xn)) * (1.0 / 3.0)
    hs_ref[...] = (l2h1_ref[...] + _l2(h3)) * 0.5


def _final(h1, h2, a, x0, l2x1, l2h1):
    return pl.pallas_call(
        _final_body,
        grid=(GRID,),
        in_specs=[pl.BlockSpec((BN, DP), lambda i: (i, 0)),
                  pl.BlockSpec((DP, DP), lambda i: (0, 0)),
                  pl.BlockSpec((BN, DP), lambda i: (i, 0)),
                  pl.BlockSpec((BN, DP), lambda i: (i, 0)),
                  pl.BlockSpec((BN, DP), lambda i: (i, 0)),
                  pl.BlockSpec((BN, DP), lambda i: (i, 0))],
        out_specs=[pl.BlockSpec((BN, DP), lambda i: (i, 0))] * 2,
        out_shape=[jax.ShapeDtypeStruct((NP, DP), _f32)] * 2,
    )(h1, h2, a, x0, l2x1, l2h1)


def kernel(embedding, adj, edge_vals, W_item0, W_item1, W_i1, W_i2,
           edge_index, channel):
    x0 = jnp.pad(embedding, ((0, NP - N), (0, DP - D)))
    adjt = jnp.pad(adj, ((0, DP - K), (0, NP - N))).T
    w0 = jnp.pad(W_item0, ((0, DP - D), (0, DP - D)))
    w1 = jnp.pad(W_item1, ((0, DP - D), (0, DP - D)))
    wi1 = jnp.pad(W_i1, ((0, DP - D), (0, DP - D))) + jnp.eye(DP, dtype=_f32)
    wi2 = jnp.pad(W_i2, ((0, DP - D), (0, DP - K)))
    row = edge_index[0]
    col = edge_index[1]

    spmm = _spmm_call()
    t1 = _matmul(x0, w0)
    xa1 = spmm(row, col, edge_vals, t1)
    h11, h21 = _cluster(xa1, adjt, wi1, wi2)
    xn1, l2x1, l2h1 = _post(h11, h21, xa1)

    t2 = _matmul(xn1, w1)
    xa2 = spmm(row, col, edge_vals, t2)
    h12, h22 = _cluster(xa2, adjt, wi1, wi2)
    item, hs = _final(h12, h22, xa2, x0, l2x1, l2h1)

    return item[:N, :D], hs[:N, :D]
